# baseline (speedup 1.0000x reference)
import functools

import jax
import jax.numpy as jnp
import numpy as np
from jax import lax
from jax.experimental import pallas as pl
from jax.experimental.pallas import tpu as pltpu
from jax.experimental.pallas import tpu_sc as plsc

N = 10000
R = 8
D = 128
E = 320000
RN = R * N

NC = 2
NS = 16
L = 16

EDGES_PER_CORE = E // NC
EDGES_PER_TILE = EDGES_PER_CORE // NS
W = 80
NWIN = EDGES_PER_TILE // W

_MESH = plsc.VectorSubcoreMesh(core_axis_name="c", subcore_axis_name="s")
_SC_PARAMS = pltpu.CompilerParams(use_tc_tiling_on_sc=False)
_SC_PARAMS_NL = pltpu.CompilerParams(use_tc_tiling_on_sc=False,
                                     needs_layout_passes=False)


CNT_ROWS_PER_TILE = RN // NS
CNT_ZROWS = 1000


@functools.partial(
    pl.kernel,
    out_type=jax.ShapeDtypeStruct((NC, RN), jnp.float32),
    mesh=_MESH,
    scratch_types=[
        pltpu.VMEM((W,), jnp.int32),
        pltpu.VMEM((W,), jnp.int32),
        pltpu.VMEM((W,), jnp.int32),
        pltpu.SemaphoreType.DMA,
        pltpu.SemaphoreType.DMA,
        pltpu.VMEM((W,), jnp.int32),
        pltpu.VMEM((W,), jnp.int32),
        pltpu.VMEM((W,), jnp.int32),
        pltpu.SemaphoreType.DMA,
        pltpu.SemaphoreType.DMA,
        pltpu.VMEM((W,), jnp.float32),
        pltpu.VMEM((CNT_ROWS_PER_TILE + 8,), jnp.float32),
        pltpu.VMEM_SHARED((RN,), jnp.float32),
    ],
    compiler_params=_SC_PARAMS,
)
def _sc_count(dst_hbm, rel_hbm, out_hbm,
              dst_a, rel_a, idx_a, sa1, sa2,
              dst_b, rel_b, idx_b, sb1, sb2,
              ones_v, zbuf, acc_sh):
    buf_a = (dst_a, rel_a, idx_a, sa1, sa2)
    buf_b = (dst_b, rel_b, idx_b, sb1, sb2)
    core = lax.axis_index("c")
    sid = lax.axis_index("s")

    @pl.loop(0, W, step=L)
    def _(k):
        ones_v[pl.ds(k, L)] = jnp.ones((L,), jnp.float32)

    @pl.loop(0, CNT_ROWS_PER_TILE + 8, step=L)
    def _(i):
        zbuf[pl.ds(i, L)] = jnp.zeros((L,), jnp.float32)

    rowstart = sid * CNT_ROWS_PER_TILE
    pltpu.sync_copy(zbuf.at[pl.ds(0, CNT_ROWS_PER_TILE)],
                    acc_sh.at[pl.ds(rowstart, CNT_ROWS_PER_TILE)])
    plsc.subcore_barrier()

    base = core * EDGES_PER_CORE + sid * EDGES_PER_TILE

    def start(w, buf):
        dst_v, rel_v, idx_v, s1, s2 = buf
        off = base + w * W
        pltpu.async_copy(dst_hbm.at[pl.ds(off, W)], dst_v, s1)
        pltpu.async_copy(rel_hbm.at[pl.ds(off, W)], rel_v, s2)

    def finish(w, buf):
        dst_v, rel_v, idx_v, s1, s2 = buf
        off = base + w * W
        pltpu.make_async_copy(dst_hbm.at[pl.ds(off, W)], dst_v, s1).wait()
        pltpu.make_async_copy(rel_hbm.at[pl.ds(off, W)], rel_v, s2).wait()

        @pl.loop(0, W, step=L)
        def _(j):
            sl = pl.ds(j, L)
            idx_v[sl] = rel_v[sl] * N + dst_v[sl]

        pltpu.sync_copy(ones_v, acc_sh.at[idx_v], add=True)

    start(0, buf_a)

    @pl.loop(0, NWIN - 1, step=2)
    def _(w):
        start(w + 1, buf_b)
        finish(w, buf_a)
        start(w + 2, buf_a)
        finish(w + 1, buf_b)

    finish(NWIN - 1, buf_a)

    plsc.subcore_barrier()
    sl = pl.ds(rowstart, CNT_ROWS_PER_TILE)
    pltpu.sync_copy(acc_sh.at[sl], out_hbm.at[core, sl])


AGG_ROWS_PER_TILE = N // NS
AGG_ZROWS = 125


def _agg_buf_types():
    return [
        pltpu.VMEM((W,), jnp.int32),
        pltpu.VMEM((W,), jnp.int32),
        pltpu.VMEM((W,), jnp.int32),
        pltpu.VMEM((W,), jnp.int32),
        pltpu.VMEM((W,), jnp.int32),
        pltpu.VMEM((W,), jnp.int32),
        pltpu.VMEM((W, D), jnp.bfloat16),
        pltpu.VMEM((W, D), jnp.float32),
        pltpu.VMEM((W, L), jnp.float32),
        pltpu.SemaphoreType.DMA,
        pltpu.SemaphoreType.DMA,
        pltpu.SemaphoreType.DMA,
        pltpu.SemaphoreType.DMA,
        pltpu.SemaphoreType.DMA,
        pltpu.SemaphoreType.DMA,
    ]


@functools.partial(
    pl.kernel,
    out_type=jax.ShapeDtypeStruct((NC, N, D), jnp.float32),
    mesh=_MESH,
    scratch_types=_agg_buf_types() + _agg_buf_types() + [
        pltpu.VMEM((AGG_ZROWS, D), jnp.float32),
        pltpu.VMEM_SHARED((N, D), jnp.float32),
    ],
    compiler_params=_SC_PARAMS_NL,
)
def _sc_agg(y_hbm, inv_hbm, src_hbm, dst_hbm, rel_hbm, out_hbm,
            *bufs_and_more):
    buf_a = bufs_and_more[0:15]
    buf_b = bufs_and_more[15:30]
    zbuf, acc_sh = bufs_and_more[30], bufs_and_more[31]
    core = lax.axis_index("c")
    sid = lax.axis_index("s")

    @pl.loop(0, AGG_ZROWS)
    def _(i):
        @pl.loop(0, D, step=L)
        def _(j):
            zbuf[i, pl.ds(j, L)] = jnp.zeros((L,), jnp.float32)

    rowstart = sid * AGG_ROWS_PER_TILE
    for j in range(AGG_ROWS_PER_TILE // AGG_ZROWS):
        pltpu.sync_copy(zbuf, acc_sh.at[pl.ds(rowstart + j * AGG_ZROWS,
                                              AGG_ZROWS)])
    plsc.subcore_barrier()

    base = core * EDGES_PER_CORE + sid * EDGES_PER_TILE

    def start_idx(w, buf):
        src_v, dst_v, rel_v = buf[0], buf[1], buf[2]
        s_src, s_dst, s_rel = buf[9], buf[10], buf[11]
        off = base + w * W
        pltpu.async_copy(src_hbm.at[pl.ds(off, W)], src_v, s_src)
        pltpu.async_copy(dst_hbm.at[pl.ds(off, W)], dst_v, s_dst)
        pltpu.async_copy(rel_hbm.at[pl.ds(off, W)], rel_v, s_rel)

    def start_gather(w, buf, pending_scatter):
        (src_v, dst_v, rel_v, gidx_v, widx_v, sdst_v, rows_bf, rows_f, w_v,
         s_src, s_dst, s_rel, s_rows, s_w, s_sc) = buf
        off = base + w * W
        pltpu.make_async_copy(src_hbm.at[pl.ds(off, W)], src_v, s_src).wait()
        pltpu.make_async_copy(dst_hbm.at[pl.ds(off, W)], dst_v, s_dst).wait()
        pltpu.make_async_copy(rel_hbm.at[pl.ds(off, W)], rel_v, s_rel).wait()

        @pl.loop(0, W, step=L)
        def _(j):
            sl = pl.ds(j, L)
            rel16 = rel_v[sl]
            gidx_v[sl] = rel16 * N + src_v[sl]
            widx_v[sl] = rel16 * N + dst_v[sl]

        if pending_scatter:
            pltpu.make_async_copy(rows_f, acc_sh.at[sdst_v], s_sc).wait()
        pltpu.async_copy(y_hbm.at[gidx_v], rows_bf, s_rows)
        pltpu.async_copy(inv_hbm.at[widx_v], w_v, s_w)

    def finish(w, buf):
        (src_v, dst_v, rel_v, gidx_v, widx_v, sdst_v, rows_bf, rows_f, w_v,
         s_src, s_dst, s_rel, s_rows, s_w, s_sc) = buf
        pltpu.make_async_copy(y_hbm.at[gidx_v], rows_bf, s_rows).wait()
        pltpu.make_async_copy(inv_hbm.at[widx_v], w_v, s_w).wait()

        @pl.loop(0, W, step=L)
        def _(j):
            sl = pl.ds(j, L)
            sdst_v[sl] = dst_v[sl]

        @pl.loop(0, W, unroll=4)
        def _(k):
            wk = w_v[k, :]
            for j in range(D // 2 // L):
                v32 = rows_bf[k, pl.ds(j * 2 * L, 2 * L)]
                lo, hi = plsc.unpack(v32, format=plsc.PackFormat.INTERLEAVED)
                rows_f[k, pl.ds(j * 2 * L, L)] = lo * wk
                rows_f[k, pl.ds(j * 2 * L + L, L)] = hi * wk

        pltpu.async_copy(rows_f, acc_sh.at[sdst_v], s_sc, add=True)

    start_idx(0, buf_a)
    start_idx(1, buf_b)
    start_gather(0, buf_a, pending_scatter=False)
    start_gather(1, buf_b, pending_scatter=False)

    @pl.loop(0, NWIN - 3, step=2)
    def _(w):
        finish(w, buf_a)
        start_idx(w + 2, buf_a)
        finish(w + 1, buf_b)
        start_idx(w + 3, buf_b)
        start_gather(w + 2, buf_a, pending_scatter=True)
        start_gather(w + 3, buf_b, pending_scatter=True)

    finish(NWIN - 3, buf_a)
    start_idx(NWIN - 1, buf_a)
    finish(NWIN - 2, buf_b)
    start_gather(NWIN - 1, buf_a, pending_scatter=True)
    finish(NWIN - 1, buf_a)

    pltpu.make_async_copy(buf_a[7], acc_sh.at[buf_a[5]], buf_a[14]).wait()
    pltpu.make_async_copy(buf_b[7], acc_sh.at[buf_b[5]], buf_b[14]).wait()

    plsc.subcore_barrier()
    for j in range(AGG_ROWS_PER_TILE // AGG_ZROWS):
        sl = pl.ds(rowstart + j * AGG_ZROWS, AGG_ZROWS)
        pltpu.sync_copy(acc_sh.at[sl], out_hbm.at[core, sl])


NB = 5
BN = N // NB


def _transform_body(x_ref, w_ref, y_ref):
    for r in range(R):
        y_ref[r] = jnp.dot(
            x_ref[...], w_ref[r], preferred_element_type=jnp.float32
        ).astype(jnp.bfloat16)


def _tc_transform(x, rel_w):
    y = pl.pallas_call(
        _transform_body,
        grid=(NB,),
        in_specs=[
            pl.BlockSpec((BN, D), lambda b: (b, 0)),
            pl.BlockSpec((R, D, D), lambda b: (0, 0, 0)),
        ],
        out_specs=pl.BlockSpec((R, BN, D), lambda b: (0, b, 0)),
        out_shape=jax.ShapeDtypeStruct((R, N, D), jnp.bfloat16),
    )(x, rel_w)
    return y.reshape(RN, D)


def _prep_body(c0_ref, c1_ref, o_ref):
    o_ref[...] = 1.0 / jnp.maximum(c0_ref[...] + c1_ref[...], 1.0)


def _tc_prep(cnt_part):
    c = cnt_part.reshape(NC, RN // D, D)
    inv = pl.pallas_call(
        _prep_body,
        grid=(1,),
        in_specs=[
            pl.BlockSpec((RN // D, D), lambda b: (0, 0)),
            pl.BlockSpec((RN // D, D), lambda b: (0, 0)),
        ],
        out_specs=pl.BlockSpec((RN // D, D), lambda b: (0, 0)),
        out_shape=jax.ShapeDtypeStruct((RN // D, D), jnp.float32),
    )(c[0], c[1])
    return jnp.broadcast_to(inv.reshape(RN, 1), (RN, L))


def _combine_body(x_ref, rw_ref, b_ref, p0_ref, p1_ref, o_ref):
    v = jnp.dot(x_ref[...], rw_ref[...], preferred_element_type=jnp.float32)
    v = v + b_ref[...] + p0_ref[...] + p1_ref[...]
    o_ref[...] = v


def _tc_combine(x, root_w, bias, part):
    return pl.pallas_call(
        _combine_body,
        grid=(NB,),
        in_specs=[
            pl.BlockSpec((BN, D), lambda b: (b, 0)),
            pl.BlockSpec((D, D), lambda b: (0, 0)),
            pl.BlockSpec((1, D), lambda b: (0, 0)),
            pl.BlockSpec((BN, D), lambda b: (b, 0)),
            pl.BlockSpec((BN, D), lambda b: (b, 0)),
        ],
        out_specs=pl.BlockSpec((BN, D), lambda b: (b, 0)),
        out_shape=jax.ShapeDtypeStruct((N, D), jnp.float32),
    )(x, root_w, bias.reshape(1, D), part[0], part[1])


def _combine_transform_body(x_ref, rw_ref, b_ref, p0_ref, p1_ref, w2_ref,
                            x2_ref, y2_ref):
    v = jnp.dot(x_ref[...], rw_ref[...], preferred_element_type=jnp.float32)
    v = v + b_ref[...] + p0_ref[...] + p1_ref[...]
    v = jnp.maximum(v, 0.0)
    x2_ref[...] = v
    for r in range(R):
        y2_ref[r] = jnp.dot(
            v, w2_ref[r], preferred_element_type=jnp.float32
        ).astype(jnp.bfloat16)


def _tc_combine_transform(x, root_w, bias, part, rel_w2):
    x2, y2 = pl.pallas_call(
        _combine_transform_body,
        grid=(NB,),
        in_specs=[
            pl.BlockSpec((BN, D), lambda b: (b, 0)),
            pl.BlockSpec((D, D), lambda b: (0, 0)),
            pl.BlockSpec((1, D), lambda b: (0, 0)),
            pl.BlockSpec((BN, D), lambda b: (b, 0)),
            pl.BlockSpec((BN, D), lambda b: (b, 0)),
            pl.BlockSpec((R, D, D), lambda b: (0, 0, 0)),
        ],
        out_specs=[
            pl.BlockSpec((BN, D), lambda b: (b, 0)),
            pl.BlockSpec((R, BN, D), lambda b: (0, b, 0)),
        ],
        out_shape=[
            jax.ShapeDtypeStruct((N, D), jnp.float32),
            jax.ShapeDtypeStruct((R, N, D), jnp.bfloat16),
        ],
    )(x, root_w, bias.reshape(1, D), part[0], part[1], rel_w2)
    return x2, y2.reshape(RN, D)


_PERM = np.arange(D).reshape(D // (2 * L), 2, L).transpose(0, 2, 1).reshape(D)
_IPERM = np.argsort(_PERM)


def kernel(edge_index, edge_type, node_emb, rel_w1, root_w1, bias1,
           rel_w2, root_w2, bias2):
    src = edge_index[0]
    dst = edge_index[1]
    rel = edge_type

    cnt_part = _sc_count(dst, rel)
    inv = _tc_prep(cnt_part)

    y1 = _tc_transform(node_emb, rel_w1[:, :, _PERM])
    p1 = _sc_agg(y1, inv, src, dst, rel)
    x2, y2 = _tc_combine_transform(node_emb, root_w1, bias1, p1,
                                   rel_w2[:, :, _PERM])

    p2 = _sc_agg(y2, inv, src, dst, rel)
    return _tc_combine(x2, root_w2, bias2, p2)


# speedup vs baseline: 1.0703x; 1.0703x over previous
import functools

import jax
import jax.numpy as jnp
import numpy as np
from jax import lax
from jax.experimental import pallas as pl
from jax.experimental.pallas import tpu as pltpu
from jax.experimental.pallas import tpu_sc as plsc

N = 10000
R = 8
D = 128
E = 320000
RN = R * N

NC = 2
NS = 16
L = 16

EDGES_PER_CORE = E // NC
EDGES_PER_TILE = EDGES_PER_CORE // NS
W = 80
NWIN = EDGES_PER_TILE // W

_MESH = plsc.VectorSubcoreMesh(core_axis_name="c", subcore_axis_name="s")
_SC_PARAMS = pltpu.CompilerParams(use_tc_tiling_on_sc=False)
_SC_PARAMS_NL = pltpu.CompilerParams(use_tc_tiling_on_sc=False,
                                     needs_layout_passes=False)


CNT_ROWS_PER_TILE = RN // NS
CNT_ZROWS = 1000


@functools.partial(
    pl.kernel,
    out_type=jax.ShapeDtypeStruct((NC, RN), jnp.float32),
    mesh=_MESH,
    scratch_types=[
        pltpu.VMEM((W,), jnp.int32),
        pltpu.VMEM((W,), jnp.int32),
        pltpu.VMEM((W,), jnp.int32),
        pltpu.SemaphoreType.DMA,
        pltpu.SemaphoreType.DMA,
        pltpu.VMEM((W,), jnp.int32),
        pltpu.VMEM((W,), jnp.int32),
        pltpu.VMEM((W,), jnp.int32),
        pltpu.SemaphoreType.DMA,
        pltpu.SemaphoreType.DMA,
        pltpu.VMEM((W,), jnp.float32),
        pltpu.VMEM((CNT_ROWS_PER_TILE + 8,), jnp.float32),
        pltpu.VMEM_SHARED((RN,), jnp.float32),
    ],
    compiler_params=_SC_PARAMS,
)
def _sc_count(dst_hbm, rel_hbm, out_hbm,
              dst_a, rel_a, idx_a, sa1, sa2,
              dst_b, rel_b, idx_b, sb1, sb2,
              ones_v, zbuf, acc_sh):
    buf_a = (dst_a, rel_a, idx_a, sa1, sa2)
    buf_b = (dst_b, rel_b, idx_b, sb1, sb2)
    core = lax.axis_index("c")
    sid = lax.axis_index("s")

    @pl.loop(0, W, step=L)
    def _(k):
        ones_v[pl.ds(k, L)] = jnp.ones((L,), jnp.float32)

    @pl.loop(0, CNT_ROWS_PER_TILE + 8, step=L)
    def _(i):
        zbuf[pl.ds(i, L)] = jnp.zeros((L,), jnp.float32)

    rowstart = sid * CNT_ROWS_PER_TILE
    pltpu.sync_copy(zbuf.at[pl.ds(0, CNT_ROWS_PER_TILE)],
                    acc_sh.at[pl.ds(rowstart, CNT_ROWS_PER_TILE)])
    plsc.subcore_barrier()

    base = core * EDGES_PER_CORE + sid * EDGES_PER_TILE

    def start(w, buf):
        dst_v, rel_v, idx_v, s1, s2 = buf
        off = base + w * W
        pltpu.async_copy(dst_hbm.at[pl.ds(off, W)], dst_v, s1)
        pltpu.async_copy(rel_hbm.at[pl.ds(off, W)], rel_v, s2)

    def finish(w, buf):
        dst_v, rel_v, idx_v, s1, s2 = buf
        off = base + w * W
        pltpu.make_async_copy(dst_hbm.at[pl.ds(off, W)], dst_v, s1).wait()
        pltpu.make_async_copy(rel_hbm.at[pl.ds(off, W)], rel_v, s2).wait()

        @pl.loop(0, W, step=L)
        def _(j):
            sl = pl.ds(j, L)
            idx_v[sl] = rel_v[sl] * N + dst_v[sl]

        pltpu.sync_copy(ones_v, acc_sh.at[idx_v], add=True)

    start(0, buf_a)

    @pl.loop(0, NWIN - 1, step=2)
    def _(w):
        start(w + 1, buf_b)
        finish(w, buf_a)
        start(w + 2, buf_a)
        finish(w + 1, buf_b)

    finish(NWIN - 1, buf_a)

    plsc.subcore_barrier()
    sl = pl.ds(rowstart, CNT_ROWS_PER_TILE)
    pltpu.sync_copy(acc_sh.at[sl], out_hbm.at[core, sl])


AGG_ROWS_PER_TILE = N // NS
AGG_ZROWS = 125


def _agg_buf_types():
    return [
        pltpu.VMEM((W,), jnp.int32),
        pltpu.VMEM((W,), jnp.int32),
        pltpu.VMEM((W,), jnp.int32),
        pltpu.VMEM((W,), jnp.int32),
        pltpu.VMEM((W,), jnp.int32),
        pltpu.VMEM((W,), jnp.int32),
        pltpu.VMEM((W, D // 2), jnp.int32),
        pltpu.VMEM((W, D), jnp.float32),
        pltpu.VMEM((W, L), jnp.float32),
        pltpu.SemaphoreType.DMA,
        pltpu.SemaphoreType.DMA,
        pltpu.SemaphoreType.DMA,
        pltpu.SemaphoreType.DMA,
        pltpu.SemaphoreType.DMA,
        pltpu.SemaphoreType.DMA,
    ]


@functools.partial(
    pl.kernel,
    out_type=jax.ShapeDtypeStruct((NC, N, D), jnp.float32),
    mesh=_MESH,
    scratch_types=_agg_buf_types() + _agg_buf_types() + [
        pltpu.VMEM((AGG_ZROWS, D), jnp.float32),
        pltpu.VMEM_SHARED((N, D), jnp.float32),
    ],
    compiler_params=_SC_PARAMS_NL,
)
def _sc_agg(y_hbm, inv_hbm, src_hbm, dst_hbm, rel_hbm, out_hbm,
            *bufs_and_more):
    buf_a = bufs_and_more[0:15]
    buf_b = bufs_and_more[15:30]
    zbuf, acc_sh = bufs_and_more[30], bufs_and_more[31]
    core = lax.axis_index("c")
    sid = lax.axis_index("s")

    @pl.loop(0, AGG_ZROWS)
    def _(i):
        @pl.loop(0, D, step=L)
        def _(j):
            zbuf[i, pl.ds(j, L)] = jnp.zeros((L,), jnp.float32)

    rowstart = sid * AGG_ROWS_PER_TILE
    for j in range(AGG_ROWS_PER_TILE // AGG_ZROWS):
        pltpu.sync_copy(zbuf, acc_sh.at[pl.ds(rowstart + j * AGG_ZROWS,
                                              AGG_ZROWS)])
    plsc.subcore_barrier()

    base = core * EDGES_PER_CORE + sid * EDGES_PER_TILE

    def start_idx(w, buf):
        src_v, dst_v, rel_v = buf[0], buf[1], buf[2]
        s_src, s_dst, s_rel = buf[9], buf[10], buf[11]
        off = base + w * W
        pltpu.async_copy(src_hbm.at[pl.ds(off, W)], src_v, s_src)
        pltpu.async_copy(dst_hbm.at[pl.ds(off, W)], dst_v, s_dst)
        pltpu.async_copy(rel_hbm.at[pl.ds(off, W)], rel_v, s_rel)

    def start_gather(w, buf, pending_scatter):
        (src_v, dst_v, rel_v, gidx_v, widx_v, sdst_v, rows_bf, rows_f, w_v,
         s_src, s_dst, s_rel, s_rows, s_w, s_sc) = buf
        off = base + w * W
        pltpu.make_async_copy(src_hbm.at[pl.ds(off, W)], src_v, s_src).wait()
        pltpu.make_async_copy(dst_hbm.at[pl.ds(off, W)], dst_v, s_dst).wait()
        pltpu.make_async_copy(rel_hbm.at[pl.ds(off, W)], rel_v, s_rel).wait()

        @pl.loop(0, W, step=L)
        def _(j):
            sl = pl.ds(j, L)
            rel16 = rel_v[sl]
            gidx_v[sl] = rel16 * N + src_v[sl]
            widx_v[sl] = rel16 * N + dst_v[sl]

        if pending_scatter:
            pltpu.make_async_copy(rows_f, acc_sh.at[sdst_v], s_sc).wait()
        pltpu.async_copy(y_hbm.at[gidx_v], rows_bf, s_rows)
        pltpu.async_copy(inv_hbm.at[widx_v], w_v, s_w)

    def finish(w, buf):
        (src_v, dst_v, rel_v, gidx_v, widx_v, sdst_v, rows_bf, rows_f, w_v,
         s_src, s_dst, s_rel, s_rows, s_w, s_sc) = buf
        pltpu.make_async_copy(y_hbm.at[gidx_v], rows_bf, s_rows).wait()
        pltpu.make_async_copy(inv_hbm.at[widx_v], w_v, s_w).wait()

        @pl.loop(0, W, step=L)
        def _(j):
            sl = pl.ds(j, L)
            sdst_v[sl] = dst_v[sl]

        @pl.loop(0, W, unroll=4)
        def _(k):
            wk = w_v[k, :]
            for j in range(D // 2 // L):
                w32 = rows_bf[k, pl.ds(j * L, L)]
                lo = plsc.bitcast(w32 << 16, jnp.float32)
                hi = plsc.bitcast(w32 & jnp.int32(-65536), jnp.float32)
                rows_f[k, pl.ds(j * 2 * L, L)] = lo * wk
                rows_f[k, pl.ds(j * 2 * L + L, L)] = hi * wk

        pltpu.async_copy(rows_f, acc_sh.at[sdst_v], s_sc, add=True)

    start_idx(0, buf_a)
    start_idx(1, buf_b)
    start_gather(0, buf_a, pending_scatter=False)
    start_gather(1, buf_b, pending_scatter=False)

    @pl.loop(0, NWIN - 3, step=2)
    def _(w):
        finish(w, buf_a)
        start_idx(w + 2, buf_a)
        finish(w + 1, buf_b)
        start_idx(w + 3, buf_b)
        start_gather(w + 2, buf_a, pending_scatter=True)
        start_gather(w + 3, buf_b, pending_scatter=True)

    finish(NWIN - 3, buf_a)
    start_idx(NWIN - 1, buf_a)
    finish(NWIN - 2, buf_b)
    start_gather(NWIN - 1, buf_a, pending_scatter=True)
    finish(NWIN - 1, buf_a)

    pltpu.make_async_copy(buf_a[7], acc_sh.at[buf_a[5]], buf_a[14]).wait()
    pltpu.make_async_copy(buf_b[7], acc_sh.at[buf_b[5]], buf_b[14]).wait()

    plsc.subcore_barrier()
    for j in range(AGG_ROWS_PER_TILE // AGG_ZROWS):
        sl = pl.ds(rowstart + j * AGG_ZROWS, AGG_ZROWS)
        pltpu.sync_copy(acc_sh.at[sl], out_hbm.at[core, sl])


NB = 5
BN = N // NB


def _rne_bf16_bits(u):
    return (u + 0x7FFF + ((u >> 16) & 1)) & jnp.int32(-65536)


def _pack_pair_words(vlo, vhi):
    ulo = _rne_bf16_bits(lax.bitcast_convert_type(vlo, jnp.int32))
    uhi = _rne_bf16_bits(lax.bitcast_convert_type(vhi, jnp.int32))
    return lax.shift_right_logical(ulo, 16) | uhi


def _transform_body(x_ref, wlo_ref, whi_ref, y_ref):
    for r in range(R):
        vlo = jnp.dot(x_ref[...], wlo_ref[r],
                      preferred_element_type=jnp.float32)
        vhi = jnp.dot(x_ref[...], whi_ref[r],
                      preferred_element_type=jnp.float32)
        y_ref[r] = _pack_pair_words(vlo, vhi)


def _tc_transform(x, rel_w_lo, rel_w_hi):
    y = pl.pallas_call(
        _transform_body,
        grid=(NB,),
        in_specs=[
            pl.BlockSpec((BN, D), lambda b: (b, 0)),
            pl.BlockSpec((R, D, D // 2), lambda b: (0, 0, 0)),
            pl.BlockSpec((R, D, D // 2), lambda b: (0, 0, 0)),
        ],
        out_specs=pl.BlockSpec((R, BN, D // 2), lambda b: (0, b, 0)),
        out_shape=jax.ShapeDtypeStruct((R, N, D // 2), jnp.int32),
    )(x, rel_w_lo, rel_w_hi)
    return y.reshape(RN, D // 2)


def _prep_body(c0_ref, c1_ref, o_ref):
    o_ref[...] = 1.0 / jnp.maximum(c0_ref[...] + c1_ref[...], 1.0)


def _tc_prep(cnt_part):
    c = cnt_part.reshape(NC, RN // D, D)
    inv = pl.pallas_call(
        _prep_body,
        grid=(1,),
        in_specs=[
            pl.BlockSpec((RN // D, D), lambda b: (0, 0)),
            pl.BlockSpec((RN // D, D), lambda b: (0, 0)),
        ],
        out_specs=pl.BlockSpec((RN // D, D), lambda b: (0, 0)),
        out_shape=jax.ShapeDtypeStruct((RN // D, D), jnp.float32),
    )(c[0], c[1])
    return jnp.broadcast_to(inv.reshape(RN, 1), (RN, L))


def _combine_body(x_ref, rw_ref, b_ref, p0_ref, p1_ref, o_ref):
    v = jnp.dot(x_ref[...], rw_ref[...], preferred_element_type=jnp.float32)
    v = v + b_ref[...] + p0_ref[...] + p1_ref[...]
    o_ref[...] = v


def _tc_combine(x, root_w, bias, part):
    return pl.pallas_call(
        _combine_body,
        grid=(NB,),
        in_specs=[
            pl.BlockSpec((BN, D), lambda b: (b, 0)),
            pl.BlockSpec((D, D), lambda b: (0, 0)),
            pl.BlockSpec((1, D), lambda b: (0, 0)),
            pl.BlockSpec((BN, D), lambda b: (b, 0)),
            pl.BlockSpec((BN, D), lambda b: (b, 0)),
        ],
        out_specs=pl.BlockSpec((BN, D), lambda b: (b, 0)),
        out_shape=jax.ShapeDtypeStruct((N, D), jnp.float32),
    )(x, root_w, bias.reshape(1, D), part[0], part[1])


def _combine_transform_body(x_ref, rw_ref, b_ref, p0_ref, p1_ref, wlo_ref,
                            whi_ref, x2_ref, y2_ref):
    v = jnp.dot(x_ref[...], rw_ref[...], preferred_element_type=jnp.float32)
    v = v + b_ref[...] + p0_ref[...] + p1_ref[...]
    v = jnp.maximum(v, 0.0)
    x2_ref[...] = v
    for r in range(R):
        vlo = jnp.dot(v, wlo_ref[r], preferred_element_type=jnp.float32)
        vhi = jnp.dot(v, whi_ref[r], preferred_element_type=jnp.float32)
        y2_ref[r] = _pack_pair_words(vlo, vhi)


def _tc_combine_transform(x, root_w, bias, part, rel_w2_lo, rel_w2_hi):
    x2, y2 = pl.pallas_call(
        _combine_transform_body,
        grid=(NB,),
        in_specs=[
            pl.BlockSpec((BN, D), lambda b: (b, 0)),
            pl.BlockSpec((D, D), lambda b: (0, 0)),
            pl.BlockSpec((1, D), lambda b: (0, 0)),
            pl.BlockSpec((BN, D), lambda b: (b, 0)),
            pl.BlockSpec((BN, D), lambda b: (b, 0)),
            pl.BlockSpec((R, D, D // 2), lambda b: (0, 0, 0)),
            pl.BlockSpec((R, D, D // 2), lambda b: (0, 0, 0)),
        ],
        out_specs=[
            pl.BlockSpec((BN, D), lambda b: (b, 0)),
            pl.BlockSpec((R, BN, D // 2), lambda b: (0, b, 0)),
        ],
        out_shape=[
            jax.ShapeDtypeStruct((N, D), jnp.float32),
            jax.ShapeDtypeStruct((R, N, D // 2), jnp.int32),
        ],
    )(x, root_w, bias.reshape(1, D), part[0], part[1], rel_w2_lo, rel_w2_hi)
    return x2, y2.reshape(RN, D // 2)


_COLS = np.arange(D).reshape(D // (2 * L), 2, L)
_PERM_L = _COLS[:, 0, :].reshape(D // 2)
_PERM_H = _COLS[:, 1, :].reshape(D // 2)


def kernel(edge_index, edge_type, node_emb, rel_w1, root_w1, bias1,
           rel_w2, root_w2, bias2):
    src = edge_index[0]
    dst = edge_index[1]
    rel = edge_type

    cnt_part = _sc_count(dst, rel)
    inv = _tc_prep(cnt_part)

    y1 = _tc_transform(node_emb, rel_w1[:, :, _PERM_L], rel_w1[:, :, _PERM_H])
    p1 = _sc_agg(y1, inv, src, dst, rel)
    x2, y2 = _tc_combine_transform(node_emb, root_w1, bias1, p1,
                                   rel_w2[:, :, _PERM_L],
                                   rel_w2[:, :, _PERM_H])

    p2 = _sc_agg(y2, inv, src, dst, rel)
    return _tc_combine(x2, root_w2, bias2, p2)
